# baseline (speedup 1.0000x reference)
import jax
import jax.numpy as jnp
from jax import lax
from jax.experimental import pallas as pl
from jax.experimental.pallas import tpu as pltpu
from jax.experimental.pallas import tpu_sc as plsc

D_MODEL = 128
MAX_REL = 128
LENGTH = 1024
BASE = 1152
B_ROWS = 2176
B_ROWS_PAD = 2304
NWORKERS = 32
SC_ROWS_PER_TILE = B_ROWS_PAD // NWORKERS
ROWS_PER_STEP = 8
LANES = 16
SPLIT = 256


def _sc_gather_body(table_hbm, b_hbm, idx_v, rows_v, sem):
    c = lax.axis_index("c")
    s = lax.axis_index("s")
    wid = s * 2 + c
    base = wid * SC_ROWS_PER_TILE

    for o in (0, 16, 32, 48, SC_ROWS_PER_TILE - LANES):
        e = base + o + lax.iota(jnp.int32, LANES)
        idx = jnp.clip(e - BASE, -MAX_REL, MAX_REL) + MAX_REL
        idx_v[pl.ds(o, LANES)] = idx

    pltpu.async_copy(table_hbm.at[idx_v], rows_v, sem).wait()
    pltpu.sync_copy(rows_v, b_hbm.at[pl.ds(base, SC_ROWS_PER_TILE)])


def _tc_head_body(table_ref, out_ref, b_ref):
    @pl.when(pl.program_id(0) == 0)
    def _fill():
        b_ref[0:1024, :] = jnp.broadcast_to(table_ref[0:1, :], (1024, D_MODEL))
        b_ref[1024:1280, :] = table_ref[0:256, :]
        b_ref[1280:B_ROWS, :] = jnp.broadcast_to(
            table_ref[256:257, :], (B_ROWS - 1280, D_MODEL)
        )

    i0 = pl.program_id(0) * ROWS_PER_STEP
    for r in range(ROWS_PER_STEP):
        out_ref[r, :, :] = b_ref[pl.ds(BASE - (i0 + r), LENGTH), :]


def _tc_tail_body(b_ref, prev_ref, out_ref):
    del prev_ref
    i0 = SPLIT + pl.program_id(0) * ROWS_PER_STEP
    for r in range(ROWS_PER_STEP):
        out_ref[r, :, :] = b_ref[pl.ds(BASE - (i0 + r), LENGTH), :]


def kernel(length, table):
    del length
    mesh = plsc.VectorSubcoreMesh(core_axis_name="c", subcore_axis_name="s")
    b = pl.kernel(
        _sc_gather_body,
        mesh=mesh,
        out_type=jax.ShapeDtypeStruct((B_ROWS_PAD, D_MODEL), jnp.float32),
        scratch_types=[
            pltpu.VMEM((SC_ROWS_PER_TILE,), jnp.int32),
            pltpu.VMEM((SC_ROWS_PER_TILE, D_MODEL), jnp.float32),
            pltpu.SemaphoreType.DMA,
        ],
    )(table)
    head = pl.pallas_call(
        _tc_head_body,
        grid=(SPLIT // ROWS_PER_STEP,),
        in_specs=[pl.BlockSpec((2 * MAX_REL + 1, D_MODEL), lambda i: (0, 0))],
        out_specs=pl.BlockSpec(
            (ROWS_PER_STEP, LENGTH, D_MODEL), lambda i: (i, 0, 0)
        ),
        out_shape=jax.ShapeDtypeStruct((LENGTH, LENGTH, D_MODEL), jnp.float32),
        scratch_shapes=[pltpu.VMEM((B_ROWS, D_MODEL), jnp.float32)],
    )(table)
    return pl.pallas_call(
        _tc_tail_body,
        grid=((LENGTH - SPLIT) // ROWS_PER_STEP,),
        in_specs=[
            pl.BlockSpec((B_ROWS_PAD, D_MODEL), lambda i: (0, 0)),
            pl.BlockSpec(memory_space=pl.ANY),
        ],
        out_specs=pl.BlockSpec(
            (ROWS_PER_STEP, LENGTH, D_MODEL),
            lambda i: (i + SPLIT // ROWS_PER_STEP, 0, 0),
        ),
        out_shape=jax.ShapeDtypeStruct((LENGTH, LENGTH, D_MODEL), jnp.float32),
        input_output_aliases={1: 0},
    )(b, head)


# speedup vs baseline: 1.0085x; 1.0085x over previous
import jax
import jax.numpy as jnp
from jax import lax
from jax.experimental import pallas as pl
from jax.experimental.pallas import tpu as pltpu
from jax.experimental.pallas import tpu_sc as plsc

D_MODEL = 128
MAX_REL = 128
LENGTH = 1024
BASE = 1152
B_ROWS = 2176
B_ROWS_PAD = 2304
NWORKERS = 32
ROWS_PER_TILE = B_ROWS_PAD // NWORKERS
ROWS_PER_STEP = 8
LANES = 16


def _sc_gather_body(table_hbm, b_hbm, idx_v, rows_v, sem):
    c = lax.axis_index("c")
    s = lax.axis_index("s")
    wid = s * 2 + c
    base = wid * ROWS_PER_TILE

    for o in (0, 16, 32, 48, ROWS_PER_TILE - LANES):
        e = base + o + lax.iota(jnp.int32, LANES)
        idx = jnp.clip(e - BASE, -MAX_REL, MAX_REL) + MAX_REL
        idx_v[pl.ds(o, LANES)] = idx

    pltpu.async_copy(table_hbm.at[idx_v], rows_v, sem).wait()
    pltpu.sync_copy(rows_v, b_hbm.at[pl.ds(base, ROWS_PER_TILE)])


def _tc_stream_body(b_ref, out_ref):
    i0 = pl.program_id(0) * ROWS_PER_STEP
    for r in range(ROWS_PER_STEP):
        out_ref[r, :, :] = b_ref[pl.ds(BASE - (i0 + r), LENGTH), :]


def kernel(length, table):
    del length
    mesh = plsc.VectorSubcoreMesh(core_axis_name="c", subcore_axis_name="s")
    b = pl.kernel(
        _sc_gather_body,
        mesh=mesh,
        out_type=jax.ShapeDtypeStruct((B_ROWS_PAD, D_MODEL), jnp.float32),
        scratch_types=[
            pltpu.VMEM((ROWS_PER_TILE,), jnp.int32),
            pltpu.VMEM((ROWS_PER_TILE, D_MODEL), jnp.float32),
            pltpu.SemaphoreType.DMA,
        ],
    )(table)
    return pl.pallas_call(
        _tc_stream_body,
        grid=(LENGTH // ROWS_PER_STEP,),
        in_specs=[pl.BlockSpec((B_ROWS_PAD, D_MODEL), lambda i: (0, 0))],
        out_specs=pl.BlockSpec(
            (ROWS_PER_STEP, LENGTH, D_MODEL), lambda i: (i, 0, 0)
        ),
        out_shape=jax.ShapeDtypeStruct((LENGTH, LENGTH, D_MODEL), jnp.float32),
    )(b)
